# baseline (speedup 1.0000x reference)
import jax
import jax.numpy as jnp
from jax import lax
from jax.experimental import pallas as pl
from jax.experimental.pallas import tpu as pltpu
from jax.experimental.pallas import tpu_sc as plsc

N_FACTORS = 16
BATCH = 16384
NC = 2
NS = 16
NW = NC * NS
BPW = BATCH // NW
LANES = 16
NCHUNK = BPW // LANES


def _mf_body(iu_hbm, ii_hbm, utabT_hbm, itabT_hbm, w_hbm, b_hbm, out_hbm,
             iu_v, ii_v, ba, bb, ucols, vcols, w_v, b_v, out_v, sema, semb):
    wid = lax.axis_index("s") * NC + lax.axis_index("c")
    base = wid * BPW

    pltpu.sync_copy(iu_hbm.at[pl.ds(base, BPW)], iu_v)
    pltpu.sync_copy(ii_hbm.at[pl.ds(base, BPW)], ii_v)
    pltpu.sync_copy(w_hbm, w_v)
    pltpu.sync_copy(b_hbm, b_v)
    wvec = w_v[...]
    b_b = b_v[...]

    lanes = lax.iota(jnp.int32, LANES)
    w_b = [jnp.take_along_axis(wvec, jnp.full((LANES,), f, jnp.int32), axis=0)
           for f in range(N_FACTORS)]

    def fire(idx_ref, tab, k, buf, sem):
        iv = idx_ref[pl.ds(k * LANES, LANES)]
        cc = lax.shift_left(lax.shift_right_logical(iv, 7), 7)
        for l in range(LANES):
            co = pl.multiple_of(cc[l], 128)
            pltpu.async_copy(tab.at[:, pl.ds(co, 128)], buf.at[l], sem)

    def fire_guarded(idx_ref, tab, k, buf, sem):
        @pl.when(k < NCHUNK)
        def _():
            fire(idx_ref, tab, k, buf, sem)

    def drain(tab, buf, sem):
        d = pltpu.make_async_copy(tab.at[:, pl.ds(0, 128)], buf.at[0], sem)
        for _ in range(LANES):
            d.wait()

    def extract(idx_ref, cols, k, buf):
        lv = lax.bitwise_and(idx_ref[pl.ds(k * LANES, LANES)], 127)
        for f in range(N_FACTORS):
            fidx = jnp.full((LANES,), f, jnp.int32)
            cols[f, pl.ds(k * LANES, LANES)] = plsc.load_gather(
                buf, [lanes, fidx, lv])

    def gather_pass(idx_ref, tab, cols):
        fire(idx_ref, tab, 0, ba, sema)
        fire(idx_ref, tab, 1, bb, semb)

        def body(g, c):
            k0 = g * 2
            drain(tab, ba, sema)
            extract(idx_ref, cols, k0, ba)
            fire_guarded(idx_ref, tab, k0 + 2, ba, sema)
            drain(tab, bb, semb)
            extract(idx_ref, cols, k0 + 1, bb)
            fire_guarded(idx_ref, tab, k0 + 3, bb, semb)
            return c

        lax.fori_loop(0, NCHUNK // 2, body, 0)

    gather_pass(iu_v, utabT_hbm, ucols)
    gather_pass(ii_v, itabT_hbm, vcols)

    def block(g, carry):
        sl = pl.ds(g * LANES, LANES)
        acc = b_b
        for f in range(N_FACTORS):
            acc = acc + ucols[f, sl] * vcols[f, sl] * w_b[f]
        out_v[sl] = acc
        return carry

    lax.fori_loop(0, NCHUNK, block, 0)

    pltpu.sync_copy(out_v, out_hbm.at[pl.ds(base, BPW)])


@jax.jit
def _mf_call(idx_users, idx_items, user_embT, item_embT, w, b):
    mesh = plsc.VectorSubcoreMesh(core_axis_name="c", subcore_axis_name="s")
    fn = pl.kernel(
        _mf_body,
        out_type=jax.ShapeDtypeStruct((BATCH,), jnp.float32),
        mesh=mesh,
        compiler_params=pltpu.CompilerParams(needs_layout_passes=False),
        scratch_types=[
            pltpu.VMEM((BPW,), jnp.int32),
            pltpu.VMEM((BPW,), jnp.int32),
            pltpu.VMEM((LANES, N_FACTORS, 128), jnp.float32),
            pltpu.VMEM((LANES, N_FACTORS, 128), jnp.float32),
            pltpu.VMEM((N_FACTORS, BPW), jnp.float32),
            pltpu.VMEM((N_FACTORS, BPW), jnp.float32),
            pltpu.VMEM((N_FACTORS,), jnp.float32),
            pltpu.VMEM((LANES,), jnp.float32),
            pltpu.VMEM((BPW,), jnp.float32),
            pltpu.SemaphoreType.DMA,
            pltpu.SemaphoreType.DMA,
        ],
    )
    return fn(idx_users, idx_items, user_embT, item_embT, w, b)


def kernel(idx_users, idx_items, user_emb_mf, item_emb_mf, W_out, b_out):
    w_row = W_out.reshape((N_FACTORS,))
    b16 = jnp.broadcast_to(b_out.reshape(()), (LANES,))
    return _mf_call(idx_users.astype(jnp.int32), idx_items.astype(jnp.int32),
                    user_emb_mf.T, item_emb_mf.T, w_row, b16)


# speedup vs baseline: 1.0481x; 1.0481x over previous
import jax
import jax.numpy as jnp
from jax import lax
from jax.experimental import pallas as pl
from jax.experimental.pallas import tpu as pltpu
from jax.experimental.pallas import tpu_sc as plsc

N_FACTORS = 16
BATCH = 16384
NC = 2
NS = 16
NW = NC * NS
BPW = BATCH // NW
LANES = 16
NCHUNK = BPW // LANES


def _mf_body(iu_hbm, ii_hbm, utabT_hbm, itabT_hbm, w_hbm, b_hbm, out_hbm,
             iu_v, ii_v, ub0, vb0, w_v, b_v, out_v, sem0):
    wid = lax.axis_index("s") * NC + lax.axis_index("c")
    base = wid * BPW

    pltpu.sync_copy(iu_hbm.at[pl.ds(base, BPW)], iu_v)
    pltpu.sync_copy(ii_hbm.at[pl.ds(base, BPW)], ii_v)
    pltpu.sync_copy(w_hbm, w_v)
    pltpu.sync_copy(b_hbm, b_v)
    wvec = w_v[...]
    b_b = b_v[...]

    lanes = lax.iota(jnp.int32, LANES)
    w_b = [jnp.take_along_axis(wvec, jnp.full((LANES,), f, jnp.int32), axis=0)
           for f in range(N_FACTORS)]

    def fire_chunk(k, ubuf, vbuf, sem):
        iu = iu_v[pl.ds(k * LANES, LANES)]
        ii = ii_v[pl.ds(k * LANES, LANES)]
        cu = lax.shift_left(lax.shift_right_logical(iu, 7), 7)
        ci = lax.shift_left(lax.shift_right_logical(ii, 7), 7)
        for l in range(LANES):
            co_u = pl.multiple_of(cu[l], 128)
            co_i = pl.multiple_of(ci[l], 128)
            pltpu.async_copy(utabT_hbm.at[:, pl.ds(co_u, 128)], ubuf.at[l], sem)
            pltpu.async_copy(itabT_hbm.at[:, pl.ds(co_i, 128)], vbuf.at[l], sem)

    def drain(ubuf, vbuf, sem):
        du = pltpu.make_async_copy(utabT_hbm.at[:, pl.ds(0, 128)], ubuf.at[0], sem)
        dv = pltpu.make_async_copy(itabT_hbm.at[:, pl.ds(0, 128)], vbuf.at[0], sem)
        for _ in range(LANES):
            du.wait()
            dv.wait()

    def compute_chunk(k, ubuf, vbuf):
        sl = pl.ds(k * LANES, LANES)
        ul = lax.bitwise_and(iu_v[sl], 127)
        il = lax.bitwise_and(ii_v[sl], 127)
        acc = b_b
        for f in range(N_FACTORS):
            fidx = jnp.full((LANES,), f, jnp.int32)
            ucol = plsc.load_gather(ubuf, [lanes, fidx, ul])
            vcol = plsc.load_gather(vbuf, [lanes, fidx, il])
            acc = acc + ucol * vcol * w_b[f]
        out_v[sl] = acc

    def step(k, carry):
        fire_chunk(k, ub0, vb0, sem0)
        drain(ub0, vb0, sem0)
        compute_chunk(k, ub0, vb0)
        return carry

    lax.fori_loop(0, NCHUNK, step, 0)

    pltpu.sync_copy(out_v, out_hbm.at[pl.ds(base, BPW)])


@jax.jit
def _mf_call(idx_users, idx_items, user_embT, item_embT, w, b):
    mesh = plsc.VectorSubcoreMesh(core_axis_name="c", subcore_axis_name="s")
    fn = pl.kernel(
        _mf_body,
        out_type=jax.ShapeDtypeStruct((BATCH,), jnp.float32),
        mesh=mesh,
        compiler_params=pltpu.CompilerParams(needs_layout_passes=False),
        scratch_types=[
            pltpu.VMEM((BPW,), jnp.int32),
            pltpu.VMEM((BPW,), jnp.int32),
            pltpu.VMEM((LANES, N_FACTORS, 128), jnp.float32),
            pltpu.VMEM((LANES, N_FACTORS, 128), jnp.float32),
            pltpu.VMEM((N_FACTORS,), jnp.float32),
            pltpu.VMEM((LANES,), jnp.float32),
            pltpu.VMEM((BPW,), jnp.float32),
            pltpu.SemaphoreType.DMA,
        ],
    )
    return fn(idx_users, idx_items, user_embT, item_embT, w, b)


def kernel(idx_users, idx_items, user_emb_mf, item_emb_mf, W_out, b_out):
    w_row = W_out.reshape((N_FACTORS,))
    b16 = jnp.broadcast_to(b_out.reshape(()), (LANES,))
    return _mf_call(idx_users.astype(jnp.int32), idx_items.astype(jnp.int32),
                    user_emb_mf.T, item_emb_mf.T, w_row, b16)
